# baseline (speedup 1.0000x reference)
import functools

import jax
import jax.numpy as jnp
from jax import lax
from jax.experimental import pallas as pl
from jax.experimental.pallas import tpu as pltpu
from jax.experimental.pallas import tpu_sc as plsc

NC = 2
NS = 16
NW = NC * NS


def kernel(x, table):
    B, S = x.shape
    M, D = table.shape
    rows_per_w = S // NW
    CHUNK = 32
    NCH = rows_per_w // CHUNK

    mesh = plsc.VectorSubcoreMesh(core_axis_name="c", subcore_axis_name="s")

    @functools.partial(
        pl.kernel,
        mesh=mesh,
        out_type=jax.ShapeDtypeStruct((B, S, D), table.dtype),
        scratch_types=[
            pltpu.VMEM((2, CHUNK, D), table.dtype),
            pltpu.SemaphoreType.DMA((2,)),
            pltpu.SemaphoreType.DMA((2,)),
        ],
    )
    def k(tab_hbm, out_hbm, buf, in_sem, out_sem):
        wid = lax.axis_index("s") * NC + lax.axis_index("c")
        base = wid * rows_per_w

        def row0(c):
            return base + c * CHUNK

        def in_copy(c, p):
            return pltpu.make_async_copy(
                tab_hbm.at[pl.ds(row0(c), CHUNK), :],
                buf.at[p],
                in_sem.at[p],
            )

        def out_copy(c, p, b):
            return pltpu.make_async_copy(
                buf.at[p],
                out_hbm.at[b, pl.ds(row0(c), CHUNK), :],
                out_sem.at[p],
            )

        in_copy(0, 0).start()
        for c in range(NCH):
            p = c % 2
            if c + 1 < NCH:
                if c >= 1:
                    for b in range(B):
                        out_copy(c - 1, p ^ 1, b).wait()
                in_copy(c + 1, p ^ 1).start()
            in_copy(c, p).wait()
            for b in range(B):
                out_copy(c, p, b).start()
        for c in range(max(0, NCH - 2), NCH):
            for b in range(B):
                out_copy(c, c % 2, b).wait()

    return k(table)


# speedup vs baseline: 1.4908x; 1.4908x over previous
import jax
import jax.numpy as jnp
from jax.experimental import pallas as pl
from jax.experimental.pallas import tpu as pltpu


def kernel(x, table):
    B, S = x.shape
    M, D = table.shape
    sizes = [256, 512, 1024, 2048, 4352]
    assert sum(sizes) == S
    offs = [sum(sizes[:i]) for i in range(len(sizes))]
    N = len(sizes)

    def body(tab_hbm, out_hbm, buf, in_sem, out_sem):
        def in_copy(i):
            return pltpu.make_async_copy(
                tab_hbm.at[pl.ds(offs[i], sizes[i]), :],
                buf.at[pl.ds(offs[i], sizes[i]), :],
                in_sem.at[i],
            )

        def out_copy(i, b):
            return pltpu.make_async_copy(
                buf.at[pl.ds(offs[i], sizes[i]), :],
                out_hbm.at[b, pl.ds(offs[i], sizes[i]), :],
                out_sem.at[i],
            )

        for i in range(N):
            in_copy(i).start()
        for i in range(N):
            in_copy(i).wait()
            for b in range(B):
                out_copy(i, b).start()
        for i in range(N):
            for b in range(B):
                out_copy(i, b).wait()

    out = pl.pallas_call(
        body,
        in_specs=[pl.BlockSpec(memory_space=pltpu.MemorySpace.HBM)],
        out_specs=pl.BlockSpec(memory_space=pltpu.MemorySpace.HBM),
        out_shape=jax.ShapeDtypeStruct((B, S, D), table.dtype),
        scratch_shapes=[
            pltpu.VMEM((S, D), table.dtype),
            pltpu.SemaphoreType.DMA((N,)),
            pltpu.SemaphoreType.DMA((N,)),
        ],
    )(table)
    return out


# speedup vs baseline: 1.5366x; 1.0307x over previous
import jax
import jax.numpy as jnp
from jax.experimental import pallas as pl
from jax.experimental.pallas import tpu as pltpu


def kernel(x, table):
    B, S = x.shape
    M, D = table.shape
    sizes = [2048, 6144]
    assert sum(sizes) == S
    offs = [sum(sizes[:i]) for i in range(len(sizes))]
    N = len(sizes)

    def body(tab_hbm, out_hbm, buf, in_sem, out_sem):
        def in_copy(i):
            return pltpu.make_async_copy(
                tab_hbm.at[pl.ds(offs[i], sizes[i]), :],
                buf.at[pl.ds(offs[i], sizes[i]), :],
                in_sem.at[i],
            )

        def out_copy(i, b):
            return pltpu.make_async_copy(
                buf.at[pl.ds(offs[i], sizes[i]), :],
                out_hbm.at[b, pl.ds(offs[i], sizes[i]), :],
                out_sem.at[i],
            )

        for i in range(N):
            in_copy(i).start()
        for i in range(N):
            in_copy(i).wait()
            for b in range(B):
                out_copy(i, b).start()
        for i in range(N):
            for b in range(B):
                out_copy(i, b).wait()

    out = pl.pallas_call(
        body,
        in_specs=[pl.BlockSpec(memory_space=pltpu.MemorySpace.HBM)],
        out_specs=pl.BlockSpec(memory_space=pltpu.MemorySpace.HBM),
        out_shape=jax.ShapeDtypeStruct((B, S, D), table.dtype),
        scratch_shapes=[
            pltpu.VMEM((S, D), table.dtype),
            pltpu.SemaphoreType.DMA((N,)),
            pltpu.SemaphoreType.DMA((N,)),
        ],
    )(table)
    return out


# speedup vs baseline: 1.5375x; 1.0006x over previous
import jax
import jax.numpy as jnp
from jax.experimental import pallas as pl
from jax.experimental.pallas import tpu as pltpu


def kernel(x, table):
    B, S = x.shape
    M, D = table.shape
    sizes = [4096, 4096]
    assert sum(sizes) == S
    offs = [sum(sizes[:i]) for i in range(len(sizes))]
    N = len(sizes)
    BUF = max(sizes)

    def body(tab_hbm, out_hbm, buf, in_sem, out_sem):
        def in_copy(i, p):
            return pltpu.make_async_copy(
                tab_hbm.at[pl.ds(offs[i], sizes[i]), :],
                buf.at[p, pl.ds(0, sizes[i]), :],
                in_sem.at[p],
            )

        def out_copies(i, p, b):
            h = sizes[i] // 2
            return [
                pltpu.make_async_copy(
                    buf.at[p, pl.ds(k * h, h), :],
                    out_hbm.at[b, pl.ds(offs[i] + k * h, h), :],
                    out_sem.at[p],
                )
                for k in range(2)
            ]

        in_copy(0, 0).start()
        for i in range(N):
            p = i % 2
            if i + 1 < N:
                if i >= 1:
                    for b in range(B):
                        for c in out_copies(i - 1, p ^ 1, b):
                            c.wait()
                in_copy(i + 1, p ^ 1).start()
            in_copy(i, p).wait()
            for b in range(B):
                for c in out_copies(i, p, b):
                    c.start()
        for i in range(max(0, N - 2), N):
            for b in range(B):
                for c in out_copies(i, i % 2, b):
                    c.wait()

    out = pl.pallas_call(
        body,
        in_specs=[pl.BlockSpec(memory_space=pltpu.MemorySpace.HBM)],
        out_specs=pl.BlockSpec(memory_space=pltpu.MemorySpace.HBM),
        out_shape=jax.ShapeDtypeStruct((B, S, D), table.dtype),
        scratch_shapes=[
            pltpu.VMEM((2, BUF, D), table.dtype),
            pltpu.SemaphoreType.DMA((2,)),
            pltpu.SemaphoreType.DMA((2,)),
        ],
    )(table)
    return out
